# baseline (speedup 1.0000x reference)
import functools

import jax
import jax.numpy as jnp
from jax import lax
from jax.experimental import pallas as pl
from jax.experimental.pallas import tpu as pltpu
from jax.experimental.pallas import tpu_sc as plsc

_N = 10000
_D = 128
_E = 320000

_NC = 2
_NS = 16
_NW = _NC * _NS
_CHUNK = 112
_NBUF = 3
_K = 90
_G = _K // (2 * _NBUF)
_E_PAD = _NW * _K * _CHUNK
_N_PAD = 10112
_RPT = _N_PAD // _NS

_mesh = plsc.VectorSubcoreMesh(core_axis_name="c", subcore_axis_name="s")


@functools.partial(
    pl.kernel,
    mesh=_mesh,
    out_type=jax.ShapeDtypeStruct((_NC, _N_PAD, _D), jnp.float32),
    scratch_types=[
        pltpu.VMEM((2, _NBUF, 2, _CHUNK), jnp.int32),
        pltpu.VMEM((_NBUF, _CHUNK, _D), jnp.float32),
        pltpu.VMEM_SHARED((_N_PAD, _D), jnp.float32),
    ] + [pltpu.SemaphoreType.DMA] * (2 + 2 * _NBUF),
)
def _sc_aggregate(h_hbm, idx_hbm, zeros_hbm, out_hbm,
                  idx_v, rows_v, agg_sh, *sems):
    isem = sems[:2]
    gsem = sems[2:2 + _NBUF]
    ssem = sems[2 + _NBUF:]
    c = lax.axis_index("c")
    s = lax.axis_index("s")
    wid = s * _NC + c
    row0 = s * _RPT

    pltpu.sync_copy(zeros_hbm.at[pl.ds(row0, _RPT)], agg_sh.at[pl.ds(row0, _RPT)])
    plsc.subcore_barrier()

    def idx_load(slot, group, sem):
        pltpu.async_copy(idx_hbm.at[wid, pl.ds(group * _NBUF, _NBUF)],
                         idx_v.at[slot], sem)

    def idx_wait(slot, sem):
        pltpu.make_async_copy(idx_hbm.at[wid, pl.ds(0, _NBUF)],
                              idx_v.at[slot], sem).wait()

    def gather(slot, b):
        pltpu.async_copy(h_hbm.at[idx_v.at[slot, b, 0]], rows_v.at[b],
                         gsem[b])

    def gather_wait(slot, b):
        pltpu.make_async_copy(h_hbm.at[idx_v.at[slot, b, 0]], rows_v.at[b],
                              gsem[b]).wait()

    def scatter(slot, b):
        pltpu.async_copy(rows_v.at[b], agg_sh.at[idx_v.at[slot, b, 1]],
                         ssem[b], add=True)

    def scatter_wait(slot, b):
        pltpu.make_async_copy(rows_v.at[b], agg_sh.at[idx_v.at[slot, b, 1]],
                              ssem[b]).wait()

    idx_load(0, 0, isem[0])
    idx_load(1, 1, isem[1])
    idx_wait(0, isem[0])
    for b in range(_NBUF):
        gather(0, b)

    def body(u, carry):
        for b in range(_NBUF):
            gather_wait(0, b)
            scatter(0, b)
        idx_wait(1, isem[1])
        for b in range(_NBUF):
            scatter_wait(0, b)
            gather(1, b)

        @pl.when(u < _G - 1)
        def _():
            idx_load(0, 2 * u + 2, isem[0])
        for b in range(_NBUF):
            gather_wait(1, b)
            scatter(1, b)

        @pl.when(u < _G - 1)
        def _():
            idx_wait(0, isem[0])
            for b in range(_NBUF):
                scatter_wait(1, b)
                gather(0, b)
            idx_load(1, 2 * u + 3, isem[1])

        @pl.when(u == _G - 1)
        def _():
            for b in range(_NBUF):
                scatter_wait(1, b)
        return carry

    lax.fori_loop(0, _G, body, 0)
    plsc.subcore_barrier()

    pltpu.sync_copy(agg_sh.at[pl.ds(row0, _RPT)],
                    out_hbm.at[c, pl.ds(row0, _RPT)])


def _gin_mlp_kernel(x_ref, a0_ref, a1_ref, w1_ref, b1_ref, w2_ref, b2_ref,
                    o_ref):
    z = x_ref[...] + a0_ref[...] + a1_ref[...]
    z = jnp.dot(z, w1_ref[...], preferred_element_type=jnp.float32) + b1_ref[...]
    z = jnp.maximum(z, 0.0)
    z = jnp.dot(z, w2_ref[...], preferred_element_type=jnp.float32) + b2_ref[...]
    o_ref[...] = jnp.maximum(z, 0.0)


def _gin_mlp_final_kernel(x_ref, a0_ref, a1_ref, w1_ref, b1_ref, w2_ref,
                          b2_ref, wm1_ref, bm1_ref, wm2_ref, bm2_ref, o_ref):
    z = x_ref[...] + a0_ref[...] + a1_ref[...]
    z = jnp.dot(z, w1_ref[...], preferred_element_type=jnp.float32) + b1_ref[...]
    z = jnp.maximum(z, 0.0)
    z = jnp.dot(z, w2_ref[...], preferred_element_type=jnp.float32) + b2_ref[...]
    z = jnp.maximum(z, 0.0)
    m = jnp.dot(z, wm1_ref[...], preferred_element_type=jnp.float32) + bm1_ref[...]
    m = jnp.maximum(m, 0.0)
    o_ref[...] = jnp.dot(m, wm2_ref[...], preferred_element_type=jnp.float32) + bm2_ref[...]


_BR = 1000
_GRID = _N // _BR

_row_spec = pl.BlockSpec((_BR, _D), lambda i: (i, 0))


def _full(shape):
    return pl.BlockSpec(shape, lambda i, _s=shape: (0,) * len(_s))


def _gin_mlp(x, a0, a1, w1, b1, w2, b2):
    return pl.pallas_call(
        _gin_mlp_kernel,
        grid=(_GRID,),
        in_specs=[_row_spec, _row_spec, _row_spec,
                  _full((_D, _D)), _full((1, _D)), _full((_D, _D)),
                  _full((1, _D))],
        out_specs=_row_spec,
        out_shape=jax.ShapeDtypeStruct((_N, _D), jnp.float32),
    )(x, a0, a1, w1, b1.reshape(1, _D), w2, b2.reshape(1, _D))


def _gin_mlp_final(x, a0, a1, w1, b1, w2, b2, wm1, bm1, wm2, bm2):
    return pl.pallas_call(
        _gin_mlp_final_kernel,
        grid=(_GRID,),
        in_specs=[_row_spec, _row_spec, _row_spec,
                  _full((_D, _D)), _full((1, _D)), _full((_D, _D)),
                  _full((1, _D)), _full((_D, 64)), _full((1, 64)),
                  _full((64, 1)), _full((1, 1))],
        out_specs=pl.BlockSpec((_BR, 1), lambda i: (i, 0)),
        out_shape=jax.ShapeDtypeStruct((_N, 1), jnp.float32),
    )(x, a0, a1, w1, b1.reshape(1, _D), w2, b2.reshape(1, _D),
      wm1, bm1.reshape(1, 64), wm2, bm2.reshape(1, 1))


def kernel(x, edge_index, W1a, b1a, W2a, b2a, W1b, b1b, W2b, b2b,
           Wm1, bm1, Wm2, bm2):
    src = edge_index[0]
    dst = edge_index[1]
    pad = _E_PAD - _E
    pad_src = jnp.arange(pad, dtype=jnp.int32) % _N
    src_p = jnp.concatenate([src, pad_src])
    trash = _N + (jnp.arange(pad, dtype=jnp.int32) % (_N_PAD - _N))
    dst_p = jnp.concatenate([dst, trash])
    src3 = src_p.reshape(_NW, _K, _CHUNK)
    dst3 = dst_p.reshape(_NW, _K, _CHUNK)
    idx = jnp.stack([src3, dst3], axis=2)
    zeros = jnp.zeros((_N_PAD, _D), jnp.float32)

    agg = _sc_aggregate(x, idx, zeros)
    h1 = _gin_mlp(x, agg[0, :_N], agg[1, :_N], W1a, b1a, W2a, b2a)
    agg2 = _sc_aggregate(h1, idx, zeros)
    out = _gin_mlp_final(h1, agg2[0, :_N], agg2[1, :_N], W1b, b1b, W2b, b2b,
                         Wm1, bm1, Wm2, bm2)
    return out[:, 0]


# speedup vs baseline: 1.1093x; 1.1093x over previous
import functools

import jax
import jax.numpy as jnp
from jax import lax
from jax.experimental import pallas as pl
from jax.experimental.pallas import tpu as pltpu
from jax.experimental.pallas import tpu_sc as plsc

_N = 10000
_D = 128
_E = 320000

_NC = 2
_NS = 16
_NW = _NC * _NS
_CHUNK = 112
_NBUF = 3
_K = 90
_G = _K // (2 * _NBUF)
_E_PAD = _NW * _K * _CHUNK
_N_PAD = 10112
_RPT = _N_PAD // _NS

_mesh = plsc.VectorSubcoreMesh(core_axis_name="c", subcore_axis_name="s")


@functools.partial(
    pl.kernel,
    mesh=_mesh,
    out_type=jax.ShapeDtypeStruct((_NC, _N_PAD, _D), jnp.float32),
    scratch_types=[
        pltpu.VMEM((2, _NBUF, 2, _CHUNK), jnp.int32),
        pltpu.VMEM((_NBUF, _CHUNK, _D), jnp.float32),
        pltpu.VMEM_SHARED((_N_PAD, _D), jnp.float32),
    ] + [pltpu.SemaphoreType.DMA] * (2 + 2 * _NBUF),
)
def _sc_aggregate(h_hbm, idx_hbm, zeros_hbm, out_hbm,
                  idx_v, rows_v, agg_sh, *sems):
    isem = sems[:2]
    gsem = sems[2:2 + _NBUF]
    ssem = sems[2 + _NBUF:]
    c = lax.axis_index("c")
    s = lax.axis_index("s")
    wid = s * _NC + c
    row0 = s * _RPT

    pltpu.sync_copy(zeros_hbm.at[pl.ds(row0, _RPT)], agg_sh.at[pl.ds(row0, _RPT)])
    plsc.subcore_barrier()

    def idx_load(slot, group, sem):
        pltpu.async_copy(idx_hbm.at[wid, pl.ds(group * _NBUF, _NBUF)],
                         idx_v.at[slot], sem)

    def idx_wait(slot, sem):
        pltpu.make_async_copy(idx_hbm.at[wid, pl.ds(0, _NBUF)],
                              idx_v.at[slot], sem).wait()

    def gather(slot, b):
        pltpu.async_copy(h_hbm.at[idx_v.at[slot, b, 0]], rows_v.at[b],
                         gsem[b])

    def gather_wait(slot, b):
        pltpu.make_async_copy(h_hbm.at[idx_v.at[slot, b, 0]], rows_v.at[b],
                              gsem[b]).wait()

    def scatter(slot, b):
        pltpu.async_copy(rows_v.at[b], agg_sh.at[idx_v.at[slot, b, 1]],
                         ssem[b], add=True)

    def scatter_wait(slot, b):
        pltpu.make_async_copy(rows_v.at[b], agg_sh.at[idx_v.at[slot, b, 1]],
                              ssem[b]).wait()

    idx_load(0, 0, isem[0])
    idx_load(1, 1, isem[1])
    idx_wait(0, isem[0])
    for b in range(_NBUF):
        gather(0, b)

    def body(u, carry):
        for b in range(_NBUF):
            gather_wait(0, b)
            scatter(0, b)
        idx_wait(1, isem[1])
        for b in range(_NBUF):
            scatter_wait(0, b)
            gather(1, b)

        @pl.when(u < _G - 1)
        def _():
            idx_load(0, 2 * u + 2, isem[0])
        for b in range(_NBUF):
            gather_wait(1, b)
            scatter(1, b)

        @pl.when(u < _G - 1)
        def _():
            idx_wait(0, isem[0])
            for b in range(_NBUF):
                scatter_wait(1, b)
                gather(0, b)
            idx_load(1, 2 * u + 3, isem[1])

        @pl.when(u == _G - 1)
        def _():
            for b in range(_NBUF):
                scatter_wait(1, b)
        return carry

    lax.fori_loop(0, _G, body, 0)
    plsc.subcore_barrier()

    pltpu.sync_copy(agg_sh.at[pl.ds(row0, _RPT)],
                    out_hbm.at[c, pl.ds(row0, _RPT)])


def _gin_mlp_kernel(x_ref, a0_ref, a1_ref, w1_ref, b1_ref, w2_ref, b2_ref,
                    o_ref):
    z = x_ref[...] + a0_ref[0] + a1_ref[0]
    z = jnp.dot(z, w1_ref[...], preferred_element_type=jnp.float32) + b1_ref[...]
    z = jnp.maximum(z, 0.0)
    z = jnp.dot(z, w2_ref[...], preferred_element_type=jnp.float32) + b2_ref[...]
    o_ref[...] = jnp.maximum(z, 0.0)


def _gin_mlp_final_kernel(x_ref, a0_ref, a1_ref, w1_ref, b1_ref, w2_ref,
                          b2_ref, wm1_ref, bm1_ref, wm2_ref, bm2_ref, o_ref):
    z = x_ref[...] + a0_ref[0] + a1_ref[0]
    z = jnp.dot(z, w1_ref[...], preferred_element_type=jnp.float32) + b1_ref[...]
    z = jnp.maximum(z, 0.0)
    z = jnp.dot(z, w2_ref[...], preferred_element_type=jnp.float32) + b2_ref[...]
    z = jnp.maximum(z, 0.0)
    m = jnp.dot(z, wm1_ref[...], preferred_element_type=jnp.float32) + bm1_ref[...]
    m = jnp.maximum(m, 0.0)
    m = jnp.dot(m, wm2_ref[...], preferred_element_type=jnp.float32) + bm2_ref[...]
    o_ref[...] = m[:, 0]


_BR = 1000
_GRID = _N // _BR

_row_spec = pl.BlockSpec((_BR, _D), lambda i: (i, 0))
_agg0_spec = pl.BlockSpec((1, _BR, _D), lambda i: (0, i, 0))
_agg1_spec = pl.BlockSpec((1, _BR, _D), lambda i: (1, i, 0))


def _full(shape):
    return pl.BlockSpec(shape, lambda i, _s=shape: (0,) * len(_s))


def _gin_mlp(x, agg, w1, b1, w2, b2):
    return pl.pallas_call(
        _gin_mlp_kernel,
        grid=(_GRID,),
        in_specs=[_row_spec, _agg0_spec, _agg1_spec,
                  _full((_D, _D)), _full((1, _D)), _full((_D, _D)),
                  _full((1, _D))],
        out_specs=_row_spec,
        out_shape=jax.ShapeDtypeStruct((_N, _D), jnp.float32),
    )(x, agg, agg, w1, b1.reshape(1, _D), w2, b2.reshape(1, _D))


def _gin_mlp_final(x, agg, w1, b1, w2, b2, wm1, bm1, wm2, bm2):
    return pl.pallas_call(
        _gin_mlp_final_kernel,
        grid=(1,),
        in_specs=[_full((_N, _D)),
                  pl.BlockSpec((1, _N, _D), lambda i: (0, 0, 0)),
                  pl.BlockSpec((1, _N, _D), lambda i: (1, 0, 0)),
                  _full((_D, _D)), _full((1, _D)), _full((_D, _D)),
                  _full((1, _D)), _full((_D, 64)), _full((1, 64)),
                  _full((64, 1)), _full((1, 1))],
        out_specs=_full((_N,)),
        out_shape=jax.ShapeDtypeStruct((_N,), jnp.float32),
    )(x, agg, agg, w1, b1.reshape(1, _D), w2, b2.reshape(1, _D),
      wm1, bm1.reshape(1, 64), wm2, bm2.reshape(1, 1))


def kernel(x, edge_index, W1a, b1a, W2a, b2a, W1b, b1b, W2b, b2b,
           Wm1, bm1, Wm2, bm2):
    pad = _E_PAD - _E
    pad_src = jnp.arange(pad, dtype=jnp.int32) % _N
    trash = _N + (jnp.arange(pad, dtype=jnp.int32) % (_N_PAD - _N))
    pad_blk = jnp.stack([pad_src, trash])
    idx = jnp.concatenate([edge_index, pad_blk], axis=1)
    idx = idx.reshape(2, _NW, _K, _CHUNK).transpose(1, 2, 0, 3)
    zeros = jnp.zeros((_N_PAD, _D), jnp.float32)

    agg = _sc_aggregate(x, idx, zeros)
    h1 = _gin_mlp(x, agg, W1a, b1a, W2a, b2a)
    agg2 = _sc_aggregate(h1, idx, zeros)
    return _gin_mlp_final(h1, agg2, W1b, b1b, W2b, b2b, Wm1, bm1, Wm2, bm2)
